# baseline (speedup 1.0000x reference)
import functools

import jax
import jax.numpy as jnp
from jax import lax
from jax.experimental import pallas as pl
from jax.experimental.pallas import tpu as pltpu
from jax.experimental.pallas import tpu_sc as plsc

NUM_CLASSES = 100000
FEAT_DIM = 128
BATCH = 16384

NC = 2
NS = 16
L = 16
NW = NC * NS
BW = BATCH // NW
CHUNK = 128
NCHUNK = BW // CHUNK
NBUF = 3
SLICES = FEAT_DIM // L


def _sc_partials(inputs_r, labels_r, centers):
  mesh = plsc.VectorSubcoreMesh(core_axis_name="c", subcore_axis_name="s")

  @functools.partial(
      pl.kernel,
      mesh=mesh,
      out_type=jax.ShapeDtypeStruct((NW, L), jnp.float32),
      scratch_types=[
          pltpu.VMEM((NCHUNK, CHUNK), jnp.int32),
          pltpu.VMEM((NBUF, CHUNK, FEAT_DIM), jnp.float32),
          pltpu.VMEM((NBUF, CHUNK, FEAT_DIM), jnp.float32),
          pltpu.VMEM((L,), jnp.float32),
          pltpu.SemaphoreType.DMA,
          pltpu.SemaphoreType.DMA,
          pltpu.SemaphoreType.DMA,
          pltpu.SemaphoreType.DMA,
          pltpu.SemaphoreType.DMA,
          pltpu.SemaphoreType.DMA,
      ],
  )
  def k(in_hbm, lab_hbm, ctr_hbm, out_hbm, idx_v, c_v, x_v, p_v,
        sg0, sg1, sg2, si0, si1, si2):
    wid = lax.axis_index("s") * NC + lax.axis_index("c")

    gsems = (sg0, sg1, sg2)
    isems = (si0, si1, si2)

    pltpu.sync_copy(lab_hbm.at[wid], idx_v)
    gh = []
    ih = []
    for c in range(NBUF):
      gh.append(pltpu.async_copy(ctr_hbm.at[idx_v.at[c]], c_v.at[c % NBUF],
                                 gsems[c % NBUF]))
      ih.append(pltpu.async_copy(in_hbm.at[wid, c], x_v.at[c % NBUF],
                                 isems[c % NBUF]))

    def chunk_sum(buf):
      zero = jnp.zeros((L,), jnp.float32)

      @plsc.parallel_loop(0, CHUNK, unroll=2, carry=(zero,) * SLICES)
      def accs(i, accs):
        out = []
        for j in range(SLICES):
          x = x_v[buf, i, pl.ds(j * L, L)]
          c = c_v[buf, i, pl.ds(j * L, L)]
          d = x - c
          sq = jnp.minimum(jnp.maximum(d * d, 1e-12), 1e12)
          out.append(accs[j] + sq)
        return tuple(out)

      t = accs[0]
      for j in range(1, SLICES):
        t = t + accs[j]
      return t

    total = jnp.zeros((L,), jnp.float32)
    for c in range(NCHUNK):
      gh[c].wait()
      ih[c].wait()
      total = total + chunk_sum(c % NBUF)
      if c + NBUF < NCHUNK:
        gh.append(pltpu.async_copy(ctr_hbm.at[idx_v.at[c + NBUF]],
                                   c_v.at[c % NBUF], gsems[c % NBUF]))
        ih.append(pltpu.async_copy(in_hbm.at[wid, c + NBUF],
                                   x_v.at[c % NBUF], isems[c % NBUF]))

    p_v[...] = total * jnp.float32(1.0 / BATCH)
    pltpu.sync_copy(p_v, out_hbm.at[wid])

  return k(inputs_r, labels_r, centers)


def kernel(inputs, labels, centers):
  inputs_r = inputs.reshape(NW, NCHUNK, CHUNK, FEAT_DIM)
  labels_r = labels.astype(jnp.int32).reshape(NW, NCHUNK, CHUNK)
  partials = _sc_partials(inputs_r, labels_r, centers)
  return jnp.sum(partials)


# speedup vs baseline: 1.0068x; 1.0068x over previous
import functools

import jax
import jax.numpy as jnp
from jax import lax
from jax.experimental import pallas as pl
from jax.experimental.pallas import tpu as pltpu
from jax.experimental.pallas import tpu_sc as plsc

NUM_CLASSES = 100000
FEAT_DIM = 128
BATCH = 16384

NC = 2
NS = 16
L = 16
NW = NC * NS
BW = BATCH // NW
CHUNK = 128
NCHUNK = BW // CHUNK
NBUF = 3
SLICES = FEAT_DIM // L


def _sc_partials(inputs_r, labels_r, centers):
  mesh = plsc.VectorSubcoreMesh(core_axis_name="c", subcore_axis_name="s")

  @functools.partial(
      pl.kernel,
      mesh=mesh,
      out_type=jax.ShapeDtypeStruct((NW, L), jnp.float32),
      scratch_types=[
          pltpu.VMEM((NCHUNK, CHUNK), jnp.int32),
          pltpu.VMEM((NBUF, CHUNK, FEAT_DIM), jnp.float32),
          pltpu.VMEM((NBUF, CHUNK, FEAT_DIM), jnp.float32),
          pltpu.VMEM((L,), jnp.float32),
          pltpu.SemaphoreType.DMA,
          pltpu.SemaphoreType.DMA,
          pltpu.SemaphoreType.DMA,
          pltpu.SemaphoreType.DMA,
          pltpu.SemaphoreType.DMA,
          pltpu.SemaphoreType.DMA,
      ],
  )
  def k(in_hbm, lab_hbm, ctr_hbm, out_hbm, idx_v, c_v, x_v, p_v,
        sg0, sg1, sg2, si0, si1, si2):
    wid = lax.axis_index("s") * NC + lax.axis_index("c")

    gsems = (sg0, sg1, sg2)
    isems = (si0, si1, si2)

    pltpu.sync_copy(lab_hbm.at[wid], idx_v)
    gh = []
    ih = []
    for c in range(NBUF):
      gh.append(pltpu.async_copy(ctr_hbm.at[idx_v.at[c]], c_v.at[c % NBUF],
                                 gsems[c % NBUF]))
      ih.append(pltpu.async_copy(in_hbm.at[wid, c], x_v.at[c % NBUF],
                                 isems[c % NBUF]))

    def chunk_sum(buf):
      zero = jnp.zeros((L,), jnp.float32)

      @plsc.parallel_loop(0, CHUNK, unroll=4, carry=(zero,) * SLICES)
      def accs(i, accs):
        out = []
        for j in range(SLICES):
          x = x_v[buf, i, pl.ds(j * L, L)]
          c = c_v[buf, i, pl.ds(j * L, L)]
          d = x - c
          sq = jnp.minimum(jnp.maximum(d * d, 1e-12), 1e12)
          out.append(accs[j] + sq)
        return tuple(out)

      t = accs[0]
      for j in range(1, SLICES):
        t = t + accs[j]
      return t

    total = jnp.zeros((L,), jnp.float32)
    for c in range(NCHUNK):
      gh[c].wait()
      ih[c].wait()
      total = total + chunk_sum(c % NBUF)
      if c + NBUF < NCHUNK:
        gh.append(pltpu.async_copy(ctr_hbm.at[idx_v.at[c + NBUF]],
                                   c_v.at[c % NBUF], gsems[c % NBUF]))
        ih.append(pltpu.async_copy(in_hbm.at[wid, c + NBUF],
                                   x_v.at[c % NBUF], isems[c % NBUF]))

    p_v[...] = total * jnp.float32(1.0 / BATCH)
    pltpu.sync_copy(p_v, out_hbm.at[wid])

  return k(inputs_r, labels_r, centers)


def kernel(inputs, labels, centers):
  inputs_r = inputs.reshape(NW, NCHUNK, CHUNK, FEAT_DIM)
  labels_r = labels.astype(jnp.int32).reshape(NW, NCHUNK, CHUNK)
  partials = _sc_partials(inputs_r, labels_r, centers)
  return jnp.sum(partials)
